# Initial kernel scaffold; baseline (speedup 1.0000x reference)
import jax
import jax.numpy as jnp
from jax.experimental import pallas as pl


def kernel(x, token_emb, pos_emb):
    raise NotImplementedError("write your pallas kernel here")


# speedup vs baseline: 1.4288x; 1.4288x over previous
import functools

import jax
import jax.numpy as jnp
from jax import lax
from jax.experimental import pallas as pl
from jax.experimental.pallas import tpu as pltpu
from jax.experimental.pallas import tpu_sc as plsc

NC = 2
NS = 16
LANES = 16
NW = NC * NS


def _make_sc_kernel(total, maxlen, dim, chunk):
    n_chunks = total // (NW * chunk)
    n_seq = chunk // maxlen
    half = dim // LANES

    mesh = plsc.VectorSubcoreMesh(core_axis_name="c", subcore_axis_name="s")

    @functools.partial(
        pl.kernel,
        out_type=jax.ShapeDtypeStruct((total, dim), jnp.float32),
        mesh=mesh,
        scratch_types=[
            pltpu.VMEM((chunk,), jnp.int32),
            pltpu.VMEM((chunk, dim), jnp.float32),
            pltpu.VMEM((maxlen, dim), jnp.float32),
            pltpu.SemaphoreType.DMA,
        ],
        compiler_params=pltpu.CompilerParams(use_tc_tiling_on_sc=False),
    )
    def kern(idx_hbm, tok_hbm, pos_hbm, out_hbm, idx_v, rows_v, pos_v, sem):
        wid = lax.axis_index("s") * NC + lax.axis_index("c")
        base = wid * (n_chunks * chunk)

        pltpu.sync_copy(pos_hbm, pos_v)

        def do_chunk(g, carry):
            start = base + g * chunk
            pltpu.sync_copy(idx_hbm.at[pl.ds(start, chunk)], idx_v)
            pltpu.async_copy(tok_hbm.at[idx_v], rows_v, sem).wait()

            def add_pos(j, c2):
                for h in range(half):
                    pv = pos_v[j, pl.ds(h * LANES, LANES)]
                    for s in range(n_seq):
                        plsc.addupdate(
                            rows_v.at[s * maxlen + j, pl.ds(h * LANES, LANES)], pv
                        )
                return c2

            lax.fori_loop(0, maxlen, add_pos, None)
            pltpu.sync_copy(rows_v, out_hbm.at[pl.ds(start, chunk)])
            return carry

        lax.fori_loop(0, n_chunks, do_chunk, None)

    return kern


@jax.jit
def kernel(x, token_emb, pos_emb):
    batch, maxlen = x.shape
    dim = token_emb.shape[1]
    total = batch * maxlen
    chunk = 8 * maxlen
    xf = x.reshape(total).astype(jnp.int32)
    k = _make_sc_kernel(total, maxlen, dim, chunk)
    out = k(xf, token_emb, pos_emb)
    return out.reshape(batch, maxlen, dim)


# speedup vs baseline: 1.4304x; 1.0011x over previous
import functools

import jax
import jax.numpy as jnp
from jax import lax
from jax.experimental import pallas as pl
from jax.experimental.pallas import tpu as pltpu
from jax.experimental.pallas import tpu_sc as plsc

NC = 2
NS = 16
LANES = 16
NW = NC * NS


def _make_sc_kernel(batch, maxlen, dim, nseq):
    seq_per_w = batch // NW
    n_chunks = seq_per_w // nseq
    half = dim // LANES

    mesh = plsc.VectorSubcoreMesh(core_axis_name="c", subcore_axis_name="s")

    @functools.partial(
        pl.kernel,
        out_type=jax.ShapeDtypeStruct((batch, maxlen, dim), jnp.float32),
        mesh=mesh,
        scratch_types=[
            pltpu.VMEM((nseq, maxlen), jnp.int32),
            pltpu.VMEM((nseq, maxlen, dim), jnp.float32),
            pltpu.VMEM((maxlen, dim), jnp.float32),
            pltpu.SemaphoreType.DMA,
        ],
        compiler_params=pltpu.CompilerParams(use_tc_tiling_on_sc=False),
    )
    def kern(x_hbm, tok_hbm, pos_hbm, out_hbm, idx_v, rows_v, pos_v, sem):
        wid = lax.axis_index("s") * NC + lax.axis_index("c")
        seq_base = wid * seq_per_w

        pltpu.sync_copy(pos_hbm, pos_v)

        def do_chunk(g, carry):
            s0 = seq_base + g * nseq
            pltpu.sync_copy(x_hbm.at[pl.ds(s0, nseq)], idx_v)
            for j in range(nseq):
                pltpu.async_copy(tok_hbm.at[idx_v.at[j]], rows_v.at[j], sem)
            for j in range(nseq):
                pltpu.make_async_copy(
                    tok_hbm.at[idx_v.at[j]], rows_v.at[j], sem
                ).wait()

            def add_pos(j, c2):
                for h in range(half):
                    pv = pos_v[j, pl.ds(h * LANES, LANES)]
                    for s in range(nseq):
                        plsc.addupdate(
                            rows_v.at[s, j, pl.ds(h * LANES, LANES)], pv
                        )
                return c2

            lax.fori_loop(0, maxlen, add_pos, None)
            pltpu.sync_copy(rows_v, out_hbm.at[pl.ds(s0, nseq)])
            return carry

        lax.fori_loop(0, n_chunks, do_chunk, None)

    return kern


@jax.jit
def kernel(x, token_emb, pos_emb):
    batch, maxlen = x.shape
    dim = token_emb.shape[1]
    k = _make_sc_kernel(batch, maxlen, dim, nseq=8)
    return k(x.astype(jnp.int32), token_emb, pos_emb)
